# Initial kernel scaffold; baseline (speedup 1.0000x reference)
import jax
import jax.numpy as jnp
from jax.experimental import pallas as pl


def kernel(x, W_experts, b_experts, W_gate, b_gate):
    raise NotImplementedError("write your pallas kernel here")


# speedup vs baseline: 1.5409x; 1.5409x over previous
import jax
import jax.numpy as jnp
from jax import lax
from jax.experimental import pallas as pl
from jax.experimental.pallas import tpu as pltpu


def _moe_block_kernel(x_ref, wgt_ref, bg_ref, wt_ref, be_ref, out_ref):
    xb = x_ref[...]
    logits = jnp.dot(xb, wgt_ref[...], preferred_element_type=jnp.float32)
    logits = logits + bg_ref[...]
    mx = jnp.max(logits, axis=1, keepdims=True)
    iota_e = lax.broadcasted_iota(jnp.int32, logits.shape, 1)
    cand = jnp.where(logits == mx, iota_e, logits.shape[1])
    eid = jnp.min(cand, axis=1, keepdims=True)

    E = wt_ref.shape[0]
    BT = xb.shape[0]
    OUT = wt_ref.shape[2]

    def body(e, acc):
        w = wt_ref[e]
        y = jnp.dot(xb, w, preferred_element_type=jnp.float32) + be_ref[e][None, :]
        return acc + jnp.where(eid == e, y, 0.0)

    acc = lax.fori_loop(0, E, body, jnp.zeros((BT, OUT), jnp.float32))
    out_ref[...] = acc


def kernel(x, W_experts, b_experts, W_gate, b_gate):
    N, H = x.shape
    E, OUT, _ = W_experts.shape
    BT = 512
    WT = jnp.swapaxes(W_experts, 1, 2)
    WgT = W_gate.T
    bg = b_gate.reshape(1, E)

    grid = (N // BT,)
    return pl.pallas_call(
        _moe_block_kernel,
        grid=grid,
        in_specs=[
            pl.BlockSpec((BT, H), lambda b: (b, 0)),
            pl.BlockSpec((H, E), lambda b: (0, 0)),
            pl.BlockSpec((1, E), lambda b: (0, 0)),
            pl.BlockSpec((E, H, OUT), lambda b: (0, 0, 0)),
            pl.BlockSpec((E, OUT), lambda b: (0, 0)),
        ],
        out_specs=pl.BlockSpec((BT, OUT), lambda b: (b, 0)),
        out_shape=jax.ShapeDtypeStruct((N, OUT), jnp.float32),
    )(x, WgT, bg, WT, b_experts)


# speedup vs baseline: 2.5260x; 1.6393x over previous
import functools

import jax
import jax.numpy as jnp
from jax import lax
from jax.experimental import pallas as pl
from jax.experimental.pallas import tpu as pltpu
from jax.experimental.pallas import tpu_sc as plsc

_BT = 512
_BTS = 128


def _route_kernel(x_ref, wgt_ref, bg_ref, tri_ref, triu_ref, pos_ref, seid_ref):
    N, H = x_ref.shape
    E = wgt_ref.shape[1]
    x = x_ref[...]
    logits = jnp.dot(x, wgt_ref[...], preferred_element_type=jnp.float32)
    logits = logits + bg_ref[...]
    mx = jnp.max(logits, axis=1, keepdims=True)
    iota_e = lax.broadcasted_iota(jnp.int32, (N, E), 1)
    eid = jnp.min(jnp.where(logits == mx, iota_e, E), axis=1, keepdims=True)
    onehot = (iota_e == eid).astype(jnp.float32)
    counts = jnp.sum(onehot, axis=0, keepdims=True)
    ends = lax.dot_general(counts, triu_ref[...], (((1,), (0,)), ((), ())),
                           precision=lax.Precision.HIGHEST)
    offsets = ends - counts
    off_sel = jnp.sum(onehot * offsets, axis=1, keepdims=True)

    tri = tri_ref[...]
    base = jnp.zeros((1, E), jnp.float32)
    parts = []
    for b in range(N // _BT):
        oh = onehot[b * _BT:(b + 1) * _BT]
        rb = lax.dot_general(tri, oh, (((1,), (0,)), ((), ())),
                             precision=lax.Precision.HIGHEST) + base
        parts.append(jnp.sum(rb * oh, axis=1, keepdims=True))
        base = base + jnp.sum(oh, axis=0, keepdims=True)
    rank_sel = jnp.concatenate(parts, axis=0)
    pos_ref[...] = (off_sel + rank_sel).astype(jnp.int32)

    iota_p = lax.broadcasted_iota(jnp.int32, (N, 1), 0).astype(jnp.float32)
    ge = (iota_p >= ends).astype(jnp.float32)
    seid_ref[...] = jnp.sum(ge, axis=1, keepdims=True).astype(jnp.int32)


def _route(x, W_gate, b_gate):
    N, H = x.shape
    E = W_gate.shape[0]
    WgT = W_gate.T
    bg = b_gate.reshape(1, E)
    ii = lax.broadcasted_iota(jnp.int32, (_BT, _BT), 0)
    jj = lax.broadcasted_iota(jnp.int32, (_BT, _BT), 1)
    tri = (jj < ii).astype(jnp.float32)
    ie = lax.broadcasted_iota(jnp.int32, (E, E), 0)
    je = lax.broadcasted_iota(jnp.int32, (E, E), 1)
    triu = (ie <= je).astype(jnp.float32)
    return pl.pallas_call(
        _route_kernel,
        grid=(1,),
        in_specs=[
            pl.BlockSpec((N, H), lambda i: (0, 0)),
            pl.BlockSpec((H, E), lambda i: (0, 0)),
            pl.BlockSpec((1, E), lambda i: (0, 0)),
            pl.BlockSpec((_BT, _BT), lambda i: (0, 0)),
            pl.BlockSpec((E, E), lambda i: (0, 0)),
        ],
        out_specs=[
            pl.BlockSpec((N, 1), lambda i: (0, 0)),
            pl.BlockSpec((N, 1), lambda i: (0, 0)),
        ],
        out_shape=[
            jax.ShapeDtypeStruct((N, 1), jnp.int32),
            jax.ShapeDtypeStruct((N, 1), jnp.int32),
        ],
    )(x, WgT, bg, tri, triu)


def _expert_kernel(elo_ref, ehi_ref, xs_ref, seid_ref, wt_ref, be_ref, out_ref):
    b = pl.program_id(0)
    elo = elo_ref[b]
    ehi = ehi_ref[b]
    xsb = xs_ref[...]
    seid = seid_ref[...]
    BTS, OUT = out_ref.shape

    def body(e, acc):
        w = wt_ref[e]
        y = jnp.dot(xsb, w, preferred_element_type=jnp.float32) + be_ref[e][None, :]
        return acc + jnp.where(seid == e, y, 0.0)

    out_ref[...] = lax.fori_loop(elo, ehi + 1, body,
                                 jnp.zeros((BTS, OUT), jnp.float32))


def _expert_apply(e_lo, e_hi, xs, seid2, WT, b_experts):
    N, H = xs.shape
    E, _, OUT = WT.shape
    NB = N // _BTS
    return pl.pallas_call(
        _expert_kernel,
        grid=(NB,),
        in_specs=[
            pl.BlockSpec(memory_space=pltpu.SMEM),
            pl.BlockSpec(memory_space=pltpu.SMEM),
            pl.BlockSpec((_BTS, H), lambda b: (b, 0)),
            pl.BlockSpec((_BTS, 1), lambda b: (b, 0)),
            pl.BlockSpec((E, H, OUT), lambda b: (0, 0, 0)),
            pl.BlockSpec((E, OUT), lambda b: (0, 0)),
        ],
        out_specs=pl.BlockSpec((_BTS, OUT), lambda b: (b, 0)),
        out_shape=jax.ShapeDtypeStruct((N, OUT), jnp.float32),
    )(e_lo, e_hi, xs, seid2, WT, b_experts)


def _make_permute(N, H, direction):
    info = plsc.get_sparse_core_info()
    NC, NS = info.num_cores, info.num_subcores
    TPW = N // (NC * NS)
    mesh = plsc.VectorSubcoreMesh(core_axis_name="c", subcore_axis_name="s")

    @functools.partial(
        pl.kernel, mesh=mesh,
        out_type=jax.ShapeDtypeStruct((N, H), jnp.float32),
        scratch_types=[
            pltpu.VMEM((TPW,), jnp.int32),
            pltpu.VMEM((TPW, H), jnp.float32),
            pltpu.SemaphoreType.DMA,
        ],
    )
    def permute(src_hbm, pos_hbm, dst_hbm, idx_v, rows_v, sem):
        wid = lax.axis_index("s") * NC + lax.axis_index("c")
        base = wid * TPW
        pltpu.sync_copy(pos_hbm.at[pl.ds(base, TPW)], idx_v)
        if direction == "scatter":
            pltpu.sync_copy(src_hbm.at[pl.ds(base, TPW)], rows_v)
            pltpu.async_copy(rows_v, dst_hbm.at[idx_v], sem).wait()
        else:
            pltpu.async_copy(src_hbm.at[idx_v], rows_v, sem).wait()
            pltpu.sync_copy(rows_v, dst_hbm.at[pl.ds(base, TPW)])

    return permute


def kernel(x, W_experts, b_experts, W_gate, b_gate):
    N, H = x.shape
    E, OUT, _ = W_experts.shape
    pos2, seid2 = _route(x, W_gate, b_gate)
    pos = pos2.reshape(N)
    seid = seid2.reshape(N)
    e_lo = seid[0::_BTS]
    e_hi = seid[_BTS - 1::_BTS]
    xs = _make_permute(N, H, "scatter")(x, pos)
    WT = jnp.swapaxes(W_experts, 1, 2)
    outs = _expert_apply(e_lo, e_hi, xs, seid2, WT, b_experts)
    return _make_permute(N, OUT, "gather")(outs, pos)


# speedup vs baseline: 3.2190x; 1.2743x over previous
import functools

import jax
import jax.numpy as jnp
from jax import lax
from jax.experimental import pallas as pl
from jax.experimental.pallas import tpu as pltpu
from jax.experimental.pallas import tpu_sc as plsc

_BT = 512
_BTS = 128
_NE = 4


def _route_kernel(x_ref, wg_ref, bg_ref, tri_ref, triu_ref,
                  pos_ref, seid_ref, elo_ref, ehi_ref):
    N, H = x_ref.shape
    E = wg_ref.shape[0]
    x = x_ref[...]
    logits = lax.dot_general(x, wg_ref[...], (((1,), (1,)), ((), ())),
                             preferred_element_type=jnp.float32)
    logits = logits + bg_ref[...]
    mx = jnp.max(logits, axis=1, keepdims=True)
    iota_e = lax.broadcasted_iota(jnp.int32, (N, E), 1)
    eid = jnp.min(jnp.where(logits == mx, iota_e, E), axis=1, keepdims=True)
    onehot = (iota_e == eid).astype(jnp.float32)
    counts = jnp.sum(onehot, axis=0, keepdims=True)
    ends = lax.dot_general(counts, triu_ref[...], (((1,), (0,)), ((), ())),
                           precision=lax.Precision.HIGHEST)
    offsets = ends - counts

    oh16 = onehot.astype(jnp.bfloat16)
    tri = tri_ref[...]
    base = offsets
    parts = []
    for b in range(N // _BT):
        oh = oh16[b * _BT:(b + 1) * _BT]
        rb = lax.dot_general(tri, oh, (((1,), (0,)), ((), ())),
                             preferred_element_type=jnp.float32) + base
        parts.append(jnp.sum(rb * onehot[b * _BT:(b + 1) * _BT],
                             axis=1, keepdims=True))
        base = rb[_BT - 1:_BT] + oh[_BT - 1:_BT].astype(jnp.float32)
    pos = jnp.concatenate(parts, axis=0)
    pos_ref[...] = pos.astype(jnp.int32).reshape(N)

    iota_p = lax.broadcasted_iota(jnp.int32, (N, 1), 0).astype(jnp.float32)
    ge = (iota_p >= ends).astype(jnp.float32)
    seid_ref[...] = jnp.sum(ge, axis=1, keepdims=True).astype(jnp.int32)

    NB = N // _BTS
    iota_b = lax.broadcasted_iota(jnp.int32, (NB, 1), 0).astype(jnp.float32)
    lo = (iota_b * _BTS >= ends).astype(jnp.float32)
    hi = (iota_b * _BTS + (_BTS - 1) >= ends).astype(jnp.float32)
    elo_ref[...] = jnp.sum(lo, axis=1, keepdims=True).astype(jnp.int32).reshape(NB)
    ehi_ref[...] = jnp.sum(hi, axis=1, keepdims=True).astype(jnp.int32).reshape(NB)


def _route(x, W_gate, b_gate):
    N, H = x.shape
    E = W_gate.shape[0]
    NB = N // _BTS
    bg = b_gate.reshape(1, E)
    ii = lax.broadcasted_iota(jnp.int32, (_BT, _BT), 0)
    jj = lax.broadcasted_iota(jnp.int32, (_BT, _BT), 1)
    tri = (jj < ii).astype(jnp.bfloat16)
    ie = lax.broadcasted_iota(jnp.int32, (E, E), 0)
    je = lax.broadcasted_iota(jnp.int32, (E, E), 1)
    triu = (ie <= je).astype(jnp.float32)
    return pl.pallas_call(
        _route_kernel,
        grid=(1,),
        in_specs=[
            pl.BlockSpec((N, H), lambda i: (0, 0)),
            pl.BlockSpec((E, H), lambda i: (0, 0)),
            pl.BlockSpec((1, E), lambda i: (0, 0)),
            pl.BlockSpec((_BT, _BT), lambda i: (0, 0)),
            pl.BlockSpec((E, E), lambda i: (0, 0)),
        ],
        out_specs=[
            pl.BlockSpec((N,), lambda i: (0,)),
            pl.BlockSpec((N, 1), lambda i: (0, 0)),
            pl.BlockSpec((NB,), lambda i: (0,)),
            pl.BlockSpec((NB,), lambda i: (0,)),
        ],
        out_shape=[
            jax.ShapeDtypeStruct((N,), jnp.int32),
            jax.ShapeDtypeStruct((N, 1), jnp.int32),
            jax.ShapeDtypeStruct((NB,), jnp.int32),
            jax.ShapeDtypeStruct((NB,), jnp.int32),
        ],
    )(x, W_gate, bg, tri, triu)


def _expert_kernel(elo_ref, ehi_ref, xs_ref, seid_ref, w2_ref, be_ref, out_ref):
    b = pl.program_id(0)
    elo = elo_ref[b]
    ehi = ehi_ref[b]
    xsb = xs_ref[...]
    seid = seid_ref[...]
    BTS, OUT = out_ref.shape
    E = be_ref.shape[0]
    nch = (ehi - elo) // _NE + 1

    def body(c, acc):
        e0 = elo + c * _NE
        e0c = jnp.minimum(e0, E - _NE)
        wchunk = w2_ref[pl.ds(e0c * OUT, _NE * OUT), :]
        y = lax.dot_general(xsb, wchunk, (((1,), (1,)), ((), ())),
                            preferred_element_type=jnp.float32)
        for j in range(_NE):
            ej = e0c + j
            sel = jnp.logical_and(seid == ej, ej >= e0)
            acc = acc + jnp.where(sel, y[:, j * OUT:(j + 1) * OUT], 0.0)
        return acc

    acc = lax.fori_loop(0, nch, body, jnp.zeros((BTS, OUT), jnp.float32))
    iota_e = lax.broadcasted_iota(jnp.int32, (BTS, E), 1)
    onehot = (iota_e == seid).astype(jnp.float32)
    out_ref[...] = acc + jnp.dot(onehot, be_ref[...],
                                 preferred_element_type=jnp.float32)


def _expert_apply(e_lo, e_hi, xs, seid2, W2, b_experts):
    N, H = xs.shape
    E, OUT = b_experts.shape
    NB = N // _BTS
    return pl.pallas_call(
        _expert_kernel,
        grid=(NB,),
        in_specs=[
            pl.BlockSpec(memory_space=pltpu.SMEM),
            pl.BlockSpec(memory_space=pltpu.SMEM),
            pl.BlockSpec((_BTS, H), lambda b: (b, 0)),
            pl.BlockSpec((_BTS, 1), lambda b: (b, 0)),
            pl.BlockSpec(W2.shape, lambda b: (0, 0)),
            pl.BlockSpec((E, OUT), lambda b: (0, 0)),
        ],
        out_specs=pl.BlockSpec((_BTS, OUT), lambda b: (b, 0)),
        out_shape=jax.ShapeDtypeStruct((N, OUT), jnp.float32),
    )(e_lo, e_hi, xs, seid2, W2, b_experts)


def _make_permute(N, H, direction):
    info = plsc.get_sparse_core_info()
    NC, NS = info.num_cores, info.num_subcores
    TPW = N // (NC * NS)
    mesh = plsc.VectorSubcoreMesh(core_axis_name="c", subcore_axis_name="s")

    @functools.partial(
        pl.kernel, mesh=mesh,
        out_type=jax.ShapeDtypeStruct((N, H), jnp.float32),
        scratch_types=[
            pltpu.VMEM((TPW,), jnp.int32),
            pltpu.VMEM((TPW, H), jnp.float32),
            pltpu.SemaphoreType.DMA,
        ],
    )
    def permute(src_hbm, pos_hbm, dst_hbm, idx_v, rows_v, sem):
        wid = lax.axis_index("s") * NC + lax.axis_index("c")
        base = wid * TPW
        pltpu.sync_copy(pos_hbm.at[pl.ds(base, TPW)], idx_v)
        if direction == "scatter":
            pltpu.sync_copy(src_hbm.at[pl.ds(base, TPW)], rows_v)
            pltpu.async_copy(rows_v, dst_hbm.at[idx_v], sem).wait()
        else:
            pltpu.async_copy(src_hbm.at[idx_v], rows_v, sem).wait()
            pltpu.sync_copy(rows_v, dst_hbm.at[pl.ds(base, TPW)])

    return permute


def kernel(x, W_experts, b_experts, W_gate, b_gate):
    N, H = x.shape
    E, OUT, _ = W_experts.shape
    pos, seid2, e_lo, e_hi = _route(x, W_gate, b_gate)
    xs = _make_permute(N, H, "scatter")(x, pos)
    W2 = W_experts.reshape(E * OUT, H)
    outs = _expert_apply(e_lo, e_hi, xs, seid2, W2, b_experts)
    return _make_permute(N, OUT, "gather")(outs, pos)
